# baseline (speedup 1.0000x reference)
import dataclasses
import functools

import jax
import jax.numpy as jnp
import numpy as np
from jax import lax
from jax.experimental import pallas as pl
from jax.experimental.pallas import tpu as pltpu
from jax.experimental.pallas import tpu_sc as plsc

D = 256
H = 8
P = 4
GH = 200
GW = 200
N = GH * GW
TAPS = 4 * H * P
BQ = 2000
NW = 32
QPW = N // NW
CH = 10
NCH = QPW // CH
IW = 2 * TAPS

_PERM = np.concatenate([np.arange(0, 32, 2), np.arange(1, 32, 2)])
_ROWS_PERM = (np.arange(H)[:, None] * 32 + _PERM[None, :]).reshape(-1)


def _proj_kernel(x_ref, xp_ref, wve_ref, bve_ref, wvo_ref, bvo_ref,
                 wox_ref, box_ref, woy_ref, boy_ref, waw_ref, baw_ref, m_ref,
                 val_ref, idx_ref, wgt_ref):
    i = pl.program_id(0)
    xb = x_ref[...]
    q = xb + xp_ref[...]
    ve = jnp.dot(xb, wve_ref[...], preferred_element_type=jnp.float32) + bve_ref[...]
    vo = jnp.dot(xb, wvo_ref[...], preferred_element_type=jnp.float32) + bvo_ref[...]

    def rne_bf16(v):
        u = jax.lax.bitcast_convert_type(v, jnp.uint32)
        return (u + 0x7FFF + ((u >> 16) & 1)) >> 16

    val_ref[...] = rne_bf16(ve) | (rne_bf16(vo) << 16)
    ox = jnp.dot(q, wox_ref[...], preferred_element_type=jnp.float32) + box_ref[...]
    oy = jnp.dot(q, woy_ref[...], preferred_element_type=jnp.float32) + boy_ref[...]
    logits = jnp.dot(q, waw_ref[...], preferred_element_type=jnp.float32) + baw_ref[...]
    e = jnp.exp(logits)
    gs = jnp.dot(e, m_ref[...], preferred_element_type=jnp.float32)
    aw = e / gs
    qid = i * BQ + lax.broadcasted_iota(jnp.int32, (BQ, 32), 0)
    row = (qid // GW).astype(jnp.float32)
    col = (qid % GW).astype(jnp.float32)
    px = col + ox
    py = row + oy
    x0f = jnp.floor(px)
    y0f = jnp.floor(py)
    lx = px - x0f
    ly = py - y0f
    x0 = x0f.astype(jnp.int32)
    y0 = y0f.astype(jnp.int32)
    x1 = x0 + 1
    y1 = y0 + 1
    head = lax.broadcasted_iota(jnp.int32, (BQ, 32), 1) // P

    def tap(xi, yi, w):
        valid = (xi >= 0) & (xi < GW) & (yi >= 0) & (yi < GH)
        xc = jnp.clip(xi, 0, GW - 1)
        yc = jnp.clip(yi, 0, GH - 1)
        return (yc * GW + xc) * H + head, aw * w * valid.astype(jnp.float32)

    i00, w00 = tap(x0, y0, (1.0 - lx) * (1.0 - ly))
    i10, w10 = tap(x1, y0, lx * (1.0 - ly))
    i01, w01 = tap(x0, y1, (1.0 - lx) * ly)
    i11, w11 = tap(x1, y1, lx * ly)
    idx_ref[...] = jnp.concatenate([i00, i10, i01, i11], axis=1)
    wgt_ref[...] = jnp.concatenate([w00, w10, w01, w11], axis=1)


def _out_kernel(lo_ref, hi_ref, x_ref, wlo_ref, whi_ref, bout_ref, o_ref):
    o_ref[...] = (
        jnp.dot(lo_ref[...], wlo_ref[...], preferred_element_type=jnp.float32)
        + jnp.dot(hi_ref[...], whi_ref[...], preferred_element_type=jnp.float32)
        + bout_ref[...]
        + x_ref[...]
    )


def _sc_bag(val_hbm, idx_hbm, wgt_hbm, lo_hbm, hi_hbm,
            idxb0, idxb1, wgtb0, wgtb1, rows0, rows1,
            olo0, olo1, ohi0, ohi1,
            gsem0, gsem1, isem0, isem1, osem0, osem1):
    wid = lax.axis_index("s") * 2 + lax.axis_index("c")
    base = wid * QPW
    bufs = (
        (idxb0, wgtb0, rows0, olo0, ohi0, gsem0, isem0, osem0),
        (idxb1, wgtb1, rows1, olo1, ohi1, gsem1, isem1, osem1),
    )

    def in_copies(k, s):
        idxb, wgtb = bufs[s][0], bufs[s][1]
        isem = bufs[s][6]
        qb = base + k * CH
        return (
            pltpu.make_async_copy(
                idx_hbm.at[pl.ds(qb * TAPS, CH * TAPS)], idxb, isem),
            pltpu.make_async_copy(
                wgt_hbm.at[pl.ds(qb * TAPS, CH * TAPS)], wgtb, isem),
        )

    def out_copies(k, s):
        olo, ohi = bufs[s][3], bufs[s][4]
        osem = bufs[s][7]
        qb = base + k * CH
        return (
            pltpu.make_async_copy(
                olo, lo_hbm.at[pl.ds(qb * 128, CH * 128)], osem),
            pltpu.make_async_copy(
                ohi, hi_hbm.at[pl.ds(qb * 128, CH * 128)], osem),
        )

    def start(copies):
        for c in copies:
            c.start()

    def wait(copies):
        for c in copies:
            c.wait()

    def start_gath(s):
        idxb, rows, gsem = bufs[s][0], bufs[s][2], bufs[s][5]

        @pl.loop(0, CH)
        def _(j):
            pltpu.make_async_copy(
                val_hbm.at[idxb.at[pl.ds(j * TAPS, TAPS)]], rows.at[j], gsem
            ).start()

    def compute(k, s):
        idxb, wgtb, rows, olo, ohi, gsem = bufs[s][:6]
        hi_mask = jnp.full((16,), 0xFFFF0000, jnp.uint32)

        @pl.loop(0, CH)
        def _(j):
            pltpu.make_async_copy(
                val_hbm.at[idxb.at[pl.ds(j * TAPS, TAPS)]], rows.at[j], gsem
            ).wait()

            jbase = jnp.full((16,), j * TAPS, jnp.int32)

            def head_body(h, hb, ob):
                acc = [jnp.zeros((16,), jnp.float32) for _ in range(4)]
                for corner in range(4):
                    half = 2 * (corner // 2)
                    for p in range(P):
                        t = corner * 32 + h * P + p
                        w = plsc.load_gather(wgtb, [jbase + t])
                        pr = rows[j, t, ...]
                        ev = plsc.bitcast(pr << 16, jnp.float32)
                        od = plsc.bitcast(pr & hi_mask, jnp.float32)
                        acc[half] = acc[half] + w * ev
                        acc[half + 1] = acc[half + 1] + w * od
                ob[pl.ds(j * 128 + hb * 32, 16)] = acc[0] + acc[2]
                ob[pl.ds(j * 128 + hb * 32 + 16, 16)] = acc[1] + acc[3]

            @pl.loop(0, 4)
            def _(hb):
                head_body(hb, hb, olo)

            @pl.loop(0, 4)
            def _(hb):
                head_body(hb + 4, hb, ohi)

    start(in_copies(0, 0))
    wait(in_copies(0, 0))
    start_gath(0)
    start(in_copies(1, 1))

    @pl.loop(0, NCH // 2)
    def _(k2):
        k = 2 * k2

        @pl.when(k >= 2)
        def _():
            wait(out_copies(k - 2, 0))

        compute(k, 0)
        start(out_copies(k, 0))
        start(in_copies(k + 2, 0))
        wait(in_copies(k + 1, 1))
        start_gath(1)

        @pl.when(k >= 1)
        def _():
            wait(out_copies(k - 1, 1))

        compute(k + 1, 1)
        start(out_copies(k + 1, 1))

        @pl.when(k + 3 < NCH)
        def _():
            start(in_copies(k + 3, 1))

        wait(in_copies(k + 2, 0))
        start_gath(0)

    wait(out_copies(NCH - 3, 0))
    compute(NCH - 1, 0)
    start(out_copies(NCH - 1, 0))
    wait(out_copies(NCH - 2, 1))
    wait(out_copies(NCH - 1, 0))


_sc_cp = pltpu.CompilerParams()
for _f, _v in (("needs_layout_passes", False), ("use_tc_tiling_on_sc", False)):
    if _f in pltpu.CompilerParams.__dataclass_fields__:
        _sc_cp = dataclasses.replace(_sc_cp, **{_f: _v})

_sc_call = pl.kernel(
    out_type=[
        jax.ShapeDtypeStruct((N * 128,), jnp.float32),
        jax.ShapeDtypeStruct((N * 128,), jnp.float32),
    ],
    compiler_params=_sc_cp,
    mesh=plsc.VectorSubcoreMesh(core_axis_name="c", subcore_axis_name="s"),
    scratch_types=[
        pltpu.VMEM((CH * TAPS,), jnp.int32),
        pltpu.VMEM((CH * TAPS,), jnp.int32),
        pltpu.VMEM((CH * TAPS,), jnp.float32),
        pltpu.VMEM((CH * TAPS,), jnp.float32),
        pltpu.VMEM((CH, TAPS, 16), jnp.uint32),
        pltpu.VMEM((CH, TAPS, 16), jnp.uint32),
        pltpu.VMEM((CH * 128,), jnp.float32),
        pltpu.VMEM((CH * 128,), jnp.float32),
        pltpu.VMEM((CH * 128,), jnp.float32),
        pltpu.VMEM((CH * 128,), jnp.float32),
        pltpu.SemaphoreType.DMA,
        pltpu.SemaphoreType.DMA,
        pltpu.SemaphoreType.DMA,
        pltpu.SemaphoreType.DMA,
        pltpu.SemaphoreType.DMA,
        pltpu.SemaphoreType.DMA,
    ],
)(_sc_bag)


def kernel(x, x_pos, W_off, b_off, W_aw, b_aw, W_val, b_val, W_out, b_out):
    x2 = x.reshape(N, D)
    xp2 = x_pos.reshape(N, D)
    wox = W_off[:, 0::2]
    box = b_off[0::2].reshape(1, 32)
    woy = W_off[:, 1::2]
    boy = b_off[1::2].reshape(1, 32)
    baw = b_aw.reshape(1, 32)
    ch = (np.arange(H)[:, None, None] * 32
          + np.arange(0, 32, 2)[None, None, :]
          + np.array([0, 1])[None, :, None])
    ce = jnp.asarray(ch[:, 0].reshape(-1))
    co = jnp.asarray(ch[:, 1].reshape(-1))
    wve, wvo = W_val[:, ce], W_val[:, co]
    bve, bvo = b_val[ce].reshape(1, 128), b_val[co].reshape(1, 128)
    m = jnp.asarray(np.kron(np.eye(H), np.ones((P, P))), jnp.float32)

    full = lambda r, c: pl.BlockSpec((r, c), lambda i: (0, 0))
    val, idx, wgt = pl.pallas_call(
        _proj_kernel,
        grid=(N // BQ,),
        in_specs=[
            pl.BlockSpec((BQ, D), lambda i: (i, 0)),
            pl.BlockSpec((BQ, D), lambda i: (i, 0)),
            full(D, 128), full(1, 128),
            full(D, 128), full(1, 128),
            full(D, 32), full(1, 32),
            full(D, 32), full(1, 32),
            full(D, 32), full(1, 32),
            full(32, 32),
        ],
        out_specs=[
            pl.BlockSpec((BQ, 128), lambda i: (i, 0)),
            pl.BlockSpec((BQ, 128), lambda i: (i, 0)),
            pl.BlockSpec((BQ, 128), lambda i: (i, 0)),
        ],
        out_shape=[
            jax.ShapeDtypeStruct((N, 128), jnp.uint32),
            jax.ShapeDtypeStruct((N, 128), jnp.int32),
            jax.ShapeDtypeStruct((N, 128), jnp.float32),
        ],
    )(x2, xp2, wve, bve, wvo, bvo, wox, box, woy, boy, W_aw, baw, m)

    lo, hi = _sc_call(
        val.reshape(N * H, 16), idx.reshape(N * TAPS), wgt.reshape(N * TAPS)
    )

    wp = W_out[_ROWS_PERM]
    out = pl.pallas_call(
        _out_kernel,
        grid=(N // BQ,),
        in_specs=[
            pl.BlockSpec((BQ, 128), lambda i: (i, 0)),
            pl.BlockSpec((BQ, 128), lambda i: (i, 0)),
            pl.BlockSpec((BQ, D), lambda i: (i, 0)),
            full(128, D), full(128, D), full(1, D),
        ],
        out_specs=pl.BlockSpec((BQ, D), lambda i: (i, 0)),
        out_shape=jax.ShapeDtypeStruct((N, D), jnp.float32),
    )(lo.reshape(N, 128), hi.reshape(N, 128), x2, wp[:128], wp[128:],
      b_out.reshape(1, D))
    return out.reshape(1, N, D)


# speedup vs baseline: 1.1948x; 1.1948x over previous
import dataclasses
import functools

import jax
import jax.numpy as jnp
import numpy as np
from jax import lax
from jax.experimental import pallas as pl
from jax.experimental.pallas import tpu as pltpu
from jax.experimental.pallas import tpu_sc as plsc

D = 256
H = 8
P = 4
GH = 200
GW = 200
N = GH * GW
TAPS = 4 * H * P
BQ = 2000
NW = 32
QPW = N // NW
CH = 10
NCH = QPW // CH
IW = 2 * TAPS

_PERM = np.concatenate([np.arange(0, 32, 2), np.arange(1, 32, 2)])
_ROWS_PERM = (np.arange(H)[:, None] * 32 + _PERM[None, :]).reshape(-1)


def _proj_kernel(x_ref, xp_ref, wve_ref, bve_ref, wvo_ref, bvo_ref,
                 wox_ref, box_ref, woy_ref, boy_ref, waw_ref, baw_ref, m_ref,
                 val_ref, idx_ref, wgt_ref):
    i = pl.program_id(0)
    xb = x_ref[...]
    q = xb + xp_ref[...]
    ve = jnp.dot(xb, wve_ref[...], preferred_element_type=jnp.float32) + bve_ref[...]
    vo = jnp.dot(xb, wvo_ref[...], preferred_element_type=jnp.float32) + bvo_ref[...]

    def rne_bf16(v):
        u = jax.lax.bitcast_convert_type(v, jnp.uint32)
        return (u + 0x7FFF + ((u >> 16) & 1)) >> 16

    val_ref[...] = rne_bf16(ve) | (rne_bf16(vo) << 16)
    ox = jnp.dot(q, wox_ref[...], preferred_element_type=jnp.float32) + box_ref[...]
    oy = jnp.dot(q, woy_ref[...], preferred_element_type=jnp.float32) + boy_ref[...]
    logits = jnp.dot(q, waw_ref[...], preferred_element_type=jnp.float32) + baw_ref[...]
    e = jnp.exp(logits)
    gs = jnp.dot(e, m_ref[...], preferred_element_type=jnp.float32)
    aw = e / gs
    qid = i * BQ + lax.broadcasted_iota(jnp.int32, (BQ, 32), 0)
    row = (qid // GW).astype(jnp.float32)
    col = (qid % GW).astype(jnp.float32)
    px = col + ox
    py = row + oy
    x0f = jnp.floor(px)
    y0f = jnp.floor(py)
    lx = px - x0f
    ly = py - y0f
    x0 = x0f.astype(jnp.int32)
    y0 = y0f.astype(jnp.int32)
    x1 = x0 + 1
    y1 = y0 + 1
    head = lax.broadcasted_iota(jnp.int32, (BQ, 32), 1) // P

    def tap(xi, yi, w):
        valid = (xi >= 0) & (xi < GW) & (yi >= 0) & (yi < GH)
        xc = jnp.clip(xi, 0, GW - 1)
        yc = jnp.clip(yi, 0, GH - 1)
        return (yc * GW + xc) * H + head, aw * w * valid.astype(jnp.float32)

    i00, w00 = tap(x0, y0, (1.0 - lx) * (1.0 - ly))
    i10, w10 = tap(x1, y0, lx * (1.0 - ly))
    i01, w01 = tap(x0, y1, (1.0 - lx) * ly)
    i11, w11 = tap(x1, y1, lx * ly)
    idx_ref[...] = jnp.concatenate([i00, i10, i01, i11], axis=1)
    wgt_ref[...] = jnp.concatenate([w00, w10, w01, w11], axis=1)


def _out_kernel(lo_ref, hi_ref, x_ref, wlo_ref, whi_ref, bout_ref, o_ref):
    o_ref[...] = (
        jnp.dot(lo_ref[...], wlo_ref[...], preferred_element_type=jnp.float32)
        + jnp.dot(hi_ref[...], whi_ref[...], preferred_element_type=jnp.float32)
        + bout_ref[...]
        + x_ref[...]
    )


def _sc_bag(val_hbm, idx_hbm, wgt_hbm, lo_hbm, hi_hbm,
            idxb0, idxb1, wgtb0, wgtb1, rows0, rows1,
            olo0, olo1, ohi0, ohi1,
            gsem0, gsem1, isem0, isem1, osem0, osem1):
    wid = lax.axis_index("s") * 2 + lax.axis_index("c")
    base = wid * QPW
    bufs = (
        (idxb0, wgtb0, rows0, olo0, ohi0, gsem0, isem0, osem0),
        (idxb1, wgtb1, rows1, olo1, ohi1, gsem1, isem1, osem1),
    )

    def in_copies(k, s):
        idxb, wgtb = bufs[s][0], bufs[s][1]
        isem = bufs[s][6]
        qb = base + k * CH
        return (
            pltpu.make_async_copy(
                idx_hbm.at[pl.ds(qb * TAPS, CH * TAPS)], idxb, isem),
            pltpu.make_async_copy(
                wgt_hbm.at[pl.ds(qb * TAPS, CH * TAPS)], wgtb, isem),
        )

    def out_copies(k, s):
        olo, ohi = bufs[s][3], bufs[s][4]
        osem = bufs[s][7]
        qb = base + k * CH
        return (
            pltpu.make_async_copy(
                olo, lo_hbm.at[pl.ds(qb * 128, CH * 128)], osem),
            pltpu.make_async_copy(
                ohi, hi_hbm.at[pl.ds(qb * 128, CH * 128)], osem),
        )

    def start(copies):
        for c in copies:
            c.start()

    def wait(copies):
        for c in copies:
            c.wait()

    def start_gath(s):
        idxb, rows, gsem = bufs[s][0], bufs[s][2], bufs[s][5]

        @pl.loop(0, CH)
        def _(j):
            pltpu.make_async_copy(
                val_hbm.at[idxb.at[pl.ds(j * TAPS, TAPS)]], rows.at[j], gsem
            ).start()

    def compute(k, s):
        idxb, wgtb, rows, olo, ohi, gsem = bufs[s][:6]
        hi_mask = jnp.full((16,), 0xFFFF0000, jnp.uint32)

        @pl.loop(0, CH)
        def _(j):
            pltpu.make_async_copy(
                val_hbm.at[idxb.at[pl.ds(j * TAPS, TAPS)]], rows.at[j], gsem
            ).wait()

            jbase = jnp.full((16,), j * TAPS, jnp.int32)

            def head_body(h, hb, ob):
                acc = [jnp.zeros((16,), jnp.float32) for _ in range(4)]
                for corner in range(4):
                    half = 2 * (corner // 2)
                    for p in range(P):
                        t = corner * 32 + h * P + p
                        w = plsc.load_gather(wgtb, [jbase + t])
                        pr = rows[j, t, ...]
                        ev = plsc.bitcast(pr << 16, jnp.float32)
                        od = plsc.bitcast(pr & hi_mask, jnp.float32)
                        acc[half] = acc[half] + w * ev
                        acc[half + 1] = acc[half + 1] + w * od
                ob[pl.ds(j * 128 + hb * 32, 16)] = acc[0] + acc[2]
                ob[pl.ds(j * 128 + hb * 32 + 16, 16)] = acc[1] + acc[3]

            @pl.loop(0, 4)
            def _(hb):
                head_body(hb, hb, olo)

            @pl.loop(0, 4)
            def _(hb):
                head_body(hb + 4, hb, ohi)

    start(in_copies(0, 0))
    wait(in_copies(0, 0))
    start_gath(0)
    start(in_copies(1, 1))

    @pl.loop(0, NCH // 2)
    def _(k2):
        k = 2 * k2
        wait(in_copies(k + 1, 1))
        start_gath(1)

        @pl.when(k >= 2)
        def _():
            wait(out_copies(k - 2, 0))

        compute(k, 0)
        start(out_copies(k, 0))
        start(in_copies(k + 2, 0))

        wait(in_copies(k + 2, 0))
        start_gath(0)

        @pl.when(k >= 1)
        def _():
            wait(out_copies(k - 1, 1))

        compute(k + 1, 1)
        start(out_copies(k + 1, 1))

        @pl.when(k + 3 < NCH)
        def _():
            start(in_copies(k + 3, 1))

    wait(out_copies(NCH - 3, 0))
    compute(NCH - 1, 0)
    start(out_copies(NCH - 1, 0))
    wait(out_copies(NCH - 2, 1))
    wait(out_copies(NCH - 1, 0))


_sc_cp = pltpu.CompilerParams()
for _f, _v in (("needs_layout_passes", False), ("use_tc_tiling_on_sc", False)):
    if _f in pltpu.CompilerParams.__dataclass_fields__:
        _sc_cp = dataclasses.replace(_sc_cp, **{_f: _v})

_sc_call = pl.kernel(
    out_type=[
        jax.ShapeDtypeStruct((N * 128,), jnp.float32),
        jax.ShapeDtypeStruct((N * 128,), jnp.float32),
    ],
    compiler_params=_sc_cp,
    mesh=plsc.VectorSubcoreMesh(core_axis_name="c", subcore_axis_name="s"),
    scratch_types=[
        pltpu.VMEM((CH * TAPS,), jnp.int32),
        pltpu.VMEM((CH * TAPS,), jnp.int32),
        pltpu.VMEM((CH * TAPS,), jnp.float32),
        pltpu.VMEM((CH * TAPS,), jnp.float32),
        pltpu.VMEM((CH, TAPS, 16), jnp.uint32),
        pltpu.VMEM((CH, TAPS, 16), jnp.uint32),
        pltpu.VMEM((CH * 128,), jnp.float32),
        pltpu.VMEM((CH * 128,), jnp.float32),
        pltpu.VMEM((CH * 128,), jnp.float32),
        pltpu.VMEM((CH * 128,), jnp.float32),
        pltpu.SemaphoreType.DMA,
        pltpu.SemaphoreType.DMA,
        pltpu.SemaphoreType.DMA,
        pltpu.SemaphoreType.DMA,
        pltpu.SemaphoreType.DMA,
        pltpu.SemaphoreType.DMA,
    ],
)(_sc_bag)


def kernel(x, x_pos, W_off, b_off, W_aw, b_aw, W_val, b_val, W_out, b_out):
    x2 = x.reshape(N, D)
    xp2 = x_pos.reshape(N, D)
    wox = W_off[:, 0::2]
    box = b_off[0::2].reshape(1, 32)
    woy = W_off[:, 1::2]
    boy = b_off[1::2].reshape(1, 32)
    baw = b_aw.reshape(1, 32)
    ch = (np.arange(H)[:, None, None] * 32
          + np.arange(0, 32, 2)[None, None, :]
          + np.array([0, 1])[None, :, None])
    ce = jnp.asarray(ch[:, 0].reshape(-1))
    co = jnp.asarray(ch[:, 1].reshape(-1))
    wve, wvo = W_val[:, ce], W_val[:, co]
    bve, bvo = b_val[ce].reshape(1, 128), b_val[co].reshape(1, 128)
    m = jnp.asarray(np.kron(np.eye(H), np.ones((P, P))), jnp.float32)

    full = lambda r, c: pl.BlockSpec((r, c), lambda i: (0, 0))
    val, idx, wgt = pl.pallas_call(
        _proj_kernel,
        grid=(N // BQ,),
        in_specs=[
            pl.BlockSpec((BQ, D), lambda i: (i, 0)),
            pl.BlockSpec((BQ, D), lambda i: (i, 0)),
            full(D, 128), full(1, 128),
            full(D, 128), full(1, 128),
            full(D, 32), full(1, 32),
            full(D, 32), full(1, 32),
            full(D, 32), full(1, 32),
            full(32, 32),
        ],
        out_specs=[
            pl.BlockSpec((BQ, 128), lambda i: (i, 0)),
            pl.BlockSpec((BQ, 128), lambda i: (i, 0)),
            pl.BlockSpec((BQ, 128), lambda i: (i, 0)),
        ],
        out_shape=[
            jax.ShapeDtypeStruct((N, 128), jnp.uint32),
            jax.ShapeDtypeStruct((N, 128), jnp.int32),
            jax.ShapeDtypeStruct((N, 128), jnp.float32),
        ],
    )(x2, xp2, wve, bve, wvo, bvo, wox, box, woy, boy, W_aw, baw, m)

    lo, hi = _sc_call(
        val.reshape(N * H, 16), idx.reshape(N * TAPS), wgt.reshape(N * TAPS)
    )

    wp = W_out[_ROWS_PERM]
    out = pl.pallas_call(
        _out_kernel,
        grid=(N // BQ,),
        in_specs=[
            pl.BlockSpec((BQ, 128), lambda i: (i, 0)),
            pl.BlockSpec((BQ, 128), lambda i: (i, 0)),
            pl.BlockSpec((BQ, D), lambda i: (i, 0)),
            full(128, D), full(128, D), full(1, D),
        ],
        out_specs=pl.BlockSpec((BQ, D), lambda i: (i, 0)),
        out_shape=jax.ShapeDtypeStruct((N, D), jnp.float32),
    )(lo.reshape(N, 128), hi.reshape(N, 128), x2, wp[:128], wp[128:],
      b_out.reshape(1, D))
    return out.reshape(1, N, D)


# speedup vs baseline: 1.2922x; 1.0815x over previous
import dataclasses
import functools

import jax
import jax.numpy as jnp
import numpy as np
from jax import lax
from jax.experimental import pallas as pl
from jax.experimental.pallas import tpu as pltpu
from jax.experimental.pallas import tpu_sc as plsc

D = 256
H = 8
P = 4
GH = 200
GW = 200
N = GH * GW
TAPS = 4 * H * P
BQ = 2000
NW = 32
QPW = N // NW
CH = 25
NCH = QPW // CH
IW = 2 * TAPS

_PERM = np.concatenate([np.arange(0, 32, 2), np.arange(1, 32, 2)])
_ROWS_PERM = (np.arange(H)[:, None] * 32 + _PERM[None, :]).reshape(-1)


def _proj_kernel(x_ref, xp_ref, wve_ref, bve_ref, wvo_ref, bvo_ref,
                 wox_ref, box_ref, woy_ref, boy_ref, waw_ref, baw_ref, m_ref,
                 val_ref, idx_ref, wgt_ref):
    i = pl.program_id(0)
    xb = x_ref[...]
    q = xb + xp_ref[...]
    ve = jnp.dot(xb, wve_ref[...], preferred_element_type=jnp.float32) + bve_ref[...]
    vo = jnp.dot(xb, wvo_ref[...], preferred_element_type=jnp.float32) + bvo_ref[...]

    def rne_bf16(v):
        u = jax.lax.bitcast_convert_type(v, jnp.uint32)
        return (u + 0x7FFF + ((u >> 16) & 1)) >> 16

    val_ref[...] = rne_bf16(ve) | (rne_bf16(vo) << 16)
    ox = jnp.dot(q, wox_ref[...], preferred_element_type=jnp.float32) + box_ref[...]
    oy = jnp.dot(q, woy_ref[...], preferred_element_type=jnp.float32) + boy_ref[...]
    logits = jnp.dot(q, waw_ref[...], preferred_element_type=jnp.float32) + baw_ref[...]
    e = jnp.exp(logits)
    gs = jnp.dot(e, m_ref[...], preferred_element_type=jnp.float32)
    aw = e / gs
    qid = i * BQ + lax.broadcasted_iota(jnp.int32, (BQ, 32), 0)
    row = (qid // GW).astype(jnp.float32)
    col = (qid % GW).astype(jnp.float32)
    px = col + ox
    py = row + oy
    x0f = jnp.floor(px)
    y0f = jnp.floor(py)
    lx = px - x0f
    ly = py - y0f
    x0 = x0f.astype(jnp.int32)
    y0 = y0f.astype(jnp.int32)
    x1 = x0 + 1
    y1 = y0 + 1
    head = lax.broadcasted_iota(jnp.int32, (BQ, 32), 1) // P

    def tap(xi, yi, w):
        valid = (xi >= 0) & (xi < GW) & (yi >= 0) & (yi < GH)
        xc = jnp.clip(xi, 0, GW - 1)
        yc = jnp.clip(yi, 0, GH - 1)
        return (yc * GW + xc) * H + head, aw * w * valid.astype(jnp.float32)

    i00, w00 = tap(x0, y0, (1.0 - lx) * (1.0 - ly))
    i10, w10 = tap(x1, y0, lx * (1.0 - ly))
    i01, w01 = tap(x0, y1, (1.0 - lx) * ly)
    i11, w11 = tap(x1, y1, lx * ly)
    idx_ref[...] = jnp.concatenate([i00, i10, i01, i11], axis=1)
    wgt_ref[...] = jnp.concatenate([w00, w10, w01, w11], axis=1)


def _out_kernel(lo_ref, hi_ref, x_ref, wlo_ref, whi_ref, bout_ref, o_ref):
    o_ref[...] = (
        jnp.dot(lo_ref[...], wlo_ref[...], preferred_element_type=jnp.float32)
        + jnp.dot(hi_ref[...], whi_ref[...], preferred_element_type=jnp.float32)
        + bout_ref[...]
        + x_ref[...]
    )


def _sc_bag(val_hbm, idx_hbm, wgt_hbm, lo_hbm, hi_hbm,
            idxb0, idxb1, wgtb0, wgtb1, rows0, rows1,
            olo0, olo1, ohi0, ohi1,
            gsem0, gsem1, isem0, isem1, osem0, osem1):
    wid = lax.axis_index("s") * 2 + lax.axis_index("c")
    base = wid * QPW
    bufs = (
        (idxb0, wgtb0, rows0, olo0, ohi0, gsem0, isem0, osem0),
        (idxb1, wgtb1, rows1, olo1, ohi1, gsem1, isem1, osem1),
    )

    def in_copies(k, s):
        idxb, wgtb = bufs[s][0], bufs[s][1]
        isem = bufs[s][6]
        qb = base + k * CH
        return (
            pltpu.make_async_copy(
                idx_hbm.at[pl.ds(qb * TAPS, CH * TAPS)], idxb, isem),
            pltpu.make_async_copy(
                wgt_hbm.at[pl.ds(qb * TAPS, CH * TAPS)], wgtb, isem),
        )

    def out_copies(k, s):
        olo, ohi = bufs[s][3], bufs[s][4]
        osem = bufs[s][7]
        qb = base + k * CH
        return (
            pltpu.make_async_copy(
                olo, lo_hbm.at[pl.ds(qb * 128, CH * 128)], osem),
            pltpu.make_async_copy(
                ohi, hi_hbm.at[pl.ds(qb * 128, CH * 128)], osem),
        )

    def start(copies):
        for c in copies:
            c.start()

    def wait(copies):
        for c in copies:
            c.wait()

    def start_gath(s):
        idxb, rows, gsem = bufs[s][0], bufs[s][2], bufs[s][5]

        @pl.loop(0, CH)
        def _(j):
            pltpu.make_async_copy(
                val_hbm.at[idxb.at[pl.ds(j * TAPS, TAPS)]], rows.at[j], gsem
            ).start()

    def compute(k, s):
        idxb, wgtb, rows, olo, ohi, gsem = bufs[s][:6]
        hi_mask = jnp.full((16,), 0xFFFF0000, jnp.uint32)

        @pl.loop(0, CH)
        def _(j):
            pltpu.make_async_copy(
                val_hbm.at[idxb.at[pl.ds(j * TAPS, TAPS)]], rows.at[j], gsem
            ).wait()

            jbase = jnp.full((16,), j * TAPS, jnp.int32)

            def head_body(h, hb, ob):
                acc = [jnp.zeros((16,), jnp.float32) for _ in range(4)]
                for corner in range(4):
                    half = 2 * (corner // 2)
                    for p in range(P):
                        t = corner * 32 + h * P + p
                        w = plsc.load_gather(wgtb, [jbase + t])
                        pr = rows[j, t, ...]
                        ev = plsc.bitcast(pr << 16, jnp.float32)
                        od = plsc.bitcast(pr & hi_mask, jnp.float32)
                        acc[half] = acc[half] + w * ev
                        acc[half + 1] = acc[half + 1] + w * od
                ob[pl.ds(j * 128 + hb * 32, 16)] = acc[0] + acc[2]
                ob[pl.ds(j * 128 + hb * 32 + 16, 16)] = acc[1] + acc[3]

            @pl.loop(0, 4)
            def _(hb):
                head_body(hb, hb, olo)

            @pl.loop(0, 4)
            def _(hb):
                head_body(hb + 4, hb, ohi)

    start(in_copies(0, 0))
    wait(in_copies(0, 0))
    start_gath(0)
    start(in_copies(1, 1))

    @pl.loop(0, NCH // 2)
    def _(k2):
        k = 2 * k2
        wait(in_copies(k + 1, 1))
        start_gath(1)

        @pl.when(k >= 2)
        def _():
            wait(out_copies(k - 2, 0))

        compute(k, 0)
        start(out_copies(k, 0))

        @pl.when(k + 2 < NCH)
        def _():
            start(in_copies(k + 2, 0))
            wait(in_copies(k + 2, 0))
            start_gath(0)

        @pl.when(k >= 1)
        def _():
            wait(out_copies(k - 1, 1))

        compute(k + 1, 1)
        start(out_copies(k + 1, 1))

        @pl.when(k + 3 < NCH)
        def _():
            start(in_copies(k + 3, 1))

    if NCH % 2:
        wait(out_copies(NCH - 3, 0))
        compute(NCH - 1, 0)
        start(out_copies(NCH - 1, 0))
        wait(out_copies(NCH - 2, 1))
        wait(out_copies(NCH - 1, 0))
    else:
        wait(out_copies(NCH - 2, 0))
        wait(out_copies(NCH - 1, 1))


_sc_cp = pltpu.CompilerParams()
for _f, _v in (("needs_layout_passes", False), ("use_tc_tiling_on_sc", False)):
    if _f in pltpu.CompilerParams.__dataclass_fields__:
        _sc_cp = dataclasses.replace(_sc_cp, **{_f: _v})

_sc_call = pl.kernel(
    out_type=[
        jax.ShapeDtypeStruct((N * 128,), jnp.float32),
        jax.ShapeDtypeStruct((N * 128,), jnp.float32),
    ],
    compiler_params=_sc_cp,
    mesh=plsc.VectorSubcoreMesh(core_axis_name="c", subcore_axis_name="s"),
    scratch_types=[
        pltpu.VMEM((CH * TAPS,), jnp.int32),
        pltpu.VMEM((CH * TAPS,), jnp.int32),
        pltpu.VMEM((CH * TAPS,), jnp.float32),
        pltpu.VMEM((CH * TAPS,), jnp.float32),
        pltpu.VMEM((CH, TAPS, 16), jnp.uint32),
        pltpu.VMEM((CH, TAPS, 16), jnp.uint32),
        pltpu.VMEM((CH * 128,), jnp.float32),
        pltpu.VMEM((CH * 128,), jnp.float32),
        pltpu.VMEM((CH * 128,), jnp.float32),
        pltpu.VMEM((CH * 128,), jnp.float32),
        pltpu.SemaphoreType.DMA,
        pltpu.SemaphoreType.DMA,
        pltpu.SemaphoreType.DMA,
        pltpu.SemaphoreType.DMA,
        pltpu.SemaphoreType.DMA,
        pltpu.SemaphoreType.DMA,
    ],
)(_sc_bag)


def kernel(x, x_pos, W_off, b_off, W_aw, b_aw, W_val, b_val, W_out, b_out):
    x2 = x.reshape(N, D)
    xp2 = x_pos.reshape(N, D)
    wox = W_off[:, 0::2]
    box = b_off[0::2].reshape(1, 32)
    woy = W_off[:, 1::2]
    boy = b_off[1::2].reshape(1, 32)
    baw = b_aw.reshape(1, 32)
    ch = (np.arange(H)[:, None, None] * 32
          + np.arange(0, 32, 2)[None, None, :]
          + np.array([0, 1])[None, :, None])
    ce = jnp.asarray(ch[:, 0].reshape(-1))
    co = jnp.asarray(ch[:, 1].reshape(-1))
    wve, wvo = W_val[:, ce], W_val[:, co]
    bve, bvo = b_val[ce].reshape(1, 128), b_val[co].reshape(1, 128)
    m = jnp.asarray(np.kron(np.eye(H), np.ones((P, P))), jnp.float32)

    full = lambda r, c: pl.BlockSpec((r, c), lambda i: (0, 0))
    val, idx, wgt = pl.pallas_call(
        _proj_kernel,
        grid=(N // BQ,),
        in_specs=[
            pl.BlockSpec((BQ, D), lambda i: (i, 0)),
            pl.BlockSpec((BQ, D), lambda i: (i, 0)),
            full(D, 128), full(1, 128),
            full(D, 128), full(1, 128),
            full(D, 32), full(1, 32),
            full(D, 32), full(1, 32),
            full(D, 32), full(1, 32),
            full(32, 32),
        ],
        out_specs=[
            pl.BlockSpec((BQ, 128), lambda i: (i, 0)),
            pl.BlockSpec((BQ, 128), lambda i: (i, 0)),
            pl.BlockSpec((BQ, 128), lambda i: (i, 0)),
        ],
        out_shape=[
            jax.ShapeDtypeStruct((N, 128), jnp.uint32),
            jax.ShapeDtypeStruct((N, 128), jnp.int32),
            jax.ShapeDtypeStruct((N, 128), jnp.float32),
        ],
    )(x2, xp2, wve, bve, wvo, bvo, wox, box, woy, boy, W_aw, baw, m)

    lo, hi = _sc_call(
        val.reshape(N * H, 16), idx.reshape(N * TAPS), wgt.reshape(N * TAPS)
    )

    wp = W_out[_ROWS_PERM]
    out = pl.pallas_call(
        _out_kernel,
        grid=(N // BQ,),
        in_specs=[
            pl.BlockSpec((BQ, 128), lambda i: (i, 0)),
            pl.BlockSpec((BQ, 128), lambda i: (i, 0)),
            pl.BlockSpec((BQ, D), lambda i: (i, 0)),
            full(128, D), full(128, D), full(1, D),
        ],
        out_specs=pl.BlockSpec((BQ, D), lambda i: (i, 0)),
        out_shape=jax.ShapeDtypeStruct((N, D), jnp.float32),
    )(lo.reshape(N, 128), hi.reshape(N, 128), x2, wp[:128], wp[128:],
      b_out.reshape(1, D))
    return out.reshape(1, N, D)


# speedup vs baseline: 1.3587x; 1.0515x over previous
import dataclasses
import functools

import jax
import jax.numpy as jnp
import numpy as np
from jax import lax
from jax.experimental import pallas as pl
from jax.experimental.pallas import tpu as pltpu
from jax.experimental.pallas import tpu_sc as plsc

D = 256
H = 8
P = 4
GH = 200
GW = 200
N = GH * GW
TAPS = 4 * H * P
BQ = 2000
NW = 32
QPW = N // NW
CH = 25
NCH = QPW // CH
IW = 2 * TAPS

_PERM = np.concatenate([np.arange(0, 32, 2), np.arange(1, 32, 2)])
_ROWS_PERM = (np.arange(H)[:, None] * 32 + _PERM[None, :]).reshape(-1)


def _proj_kernel(x_ref, xp_ref, wve_ref, bve_ref, wvo_ref, bvo_ref,
                 wox_ref, box_ref, woy_ref, boy_ref, waw_ref, baw_ref, m_ref,
                 val_ref, idx_ref, wgt_ref):
    i = pl.program_id(0)
    xb = x_ref[...]
    q = xb + xp_ref[...]
    ve = jnp.dot(xb, wve_ref[...], preferred_element_type=jnp.float32) + bve_ref[...]
    vo = jnp.dot(xb, wvo_ref[...], preferred_element_type=jnp.float32) + bvo_ref[...]

    def rne_bf16(v):
        u = jax.lax.bitcast_convert_type(v, jnp.uint32)
        return (u + 0x7FFF + ((u >> 16) & 1)) >> 16

    val_ref[...] = rne_bf16(ve) | (rne_bf16(vo) << 16)
    ox = jnp.dot(q, wox_ref[...], preferred_element_type=jnp.float32) + box_ref[...]
    oy = jnp.dot(q, woy_ref[...], preferred_element_type=jnp.float32) + boy_ref[...]
    logits = jnp.dot(q, waw_ref[...], preferred_element_type=jnp.float32) + baw_ref[...]
    e = jnp.exp(logits)
    gs = jnp.dot(e, m_ref[...], preferred_element_type=jnp.float32)
    aw = e / gs
    qid = i * BQ + lax.broadcasted_iota(jnp.int32, (BQ, 32), 0)
    row = (qid // GW).astype(jnp.float32)
    col = (qid % GW).astype(jnp.float32)
    px = col + ox
    py = row + oy
    x0f = jnp.floor(px)
    y0f = jnp.floor(py)
    lx = px - x0f
    ly = py - y0f
    x0 = x0f.astype(jnp.int32)
    y0 = y0f.astype(jnp.int32)
    x1 = x0 + 1
    y1 = y0 + 1
    head = lax.broadcasted_iota(jnp.int32, (BQ, 32), 1) // P

    def tap(xi, yi, w):
        valid = (xi >= 0) & (xi < GW) & (yi >= 0) & (yi < GH)
        xc = jnp.clip(xi, 0, GW - 1)
        yc = jnp.clip(yi, 0, GH - 1)
        return (yc * GW + xc) * H + head, aw * w * valid.astype(jnp.float32)

    i00, w00 = tap(x0, y0, (1.0 - lx) * (1.0 - ly))
    i10, w10 = tap(x1, y0, lx * (1.0 - ly))
    i01, w01 = tap(x0, y1, (1.0 - lx) * ly)
    i11, w11 = tap(x1, y1, lx * ly)
    idx_ref[...] = jnp.concatenate([i00, i10, i01, i11], axis=1)
    wgt_ref[...] = jnp.concatenate([w00, w10, w01, w11], axis=1)


def _out_kernel(lo_ref, hi_ref, x_ref, wlo_ref, whi_ref, bout_ref, o_ref):
    o_ref[...] = (
        jnp.dot(lo_ref[...], wlo_ref[...], preferred_element_type=jnp.float32)
        + jnp.dot(hi_ref[...], whi_ref[...], preferred_element_type=jnp.float32)
        + bout_ref[...]
        + x_ref[...]
    )


def _sc_bag(val_hbm, idx_hbm, wgt_hbm, lo_hbm, hi_hbm,
            idxb0, idxb1, wgtb0, wgtb1, rows0, rows1,
            olo0, olo1, ohi0, ohi1,
            gsem0, gsem1, isem0, isem1, osem0, osem1):
    wid = lax.axis_index("s") * 2 + lax.axis_index("c")
    base = wid * QPW
    bufs = (
        (idxb0, wgtb0, rows0, olo0, ohi0, gsem0, isem0, osem0),
        (idxb1, wgtb1, rows1, olo1, ohi1, gsem1, isem1, osem1),
    )

    def in_copies(k, s):
        idxb, wgtb = bufs[s][0], bufs[s][1]
        isem = bufs[s][6]
        qb = base + k * CH
        return (
            pltpu.make_async_copy(
                idx_hbm.at[pl.ds(qb * TAPS, CH * TAPS)], idxb, isem),
            pltpu.make_async_copy(
                wgt_hbm.at[pl.ds(qb * TAPS, CH * TAPS)], wgtb, isem),
        )

    def out_copies(k, s):
        olo, ohi = bufs[s][3], bufs[s][4]
        osem = bufs[s][7]
        qb = base + k * CH
        return (
            pltpu.make_async_copy(
                olo, lo_hbm.at[pl.ds(qb * 128, CH * 128)], osem),
            pltpu.make_async_copy(
                ohi, hi_hbm.at[pl.ds(qb * 128, CH * 128)], osem),
        )

    def start(copies):
        for c in copies:
            c.start()

    def wait(copies):
        for c in copies:
            c.wait()

    def start_gath(s):
        idxb, rows, gsem = bufs[s][0], bufs[s][2], bufs[s][5]

        @pl.loop(0, CH)
        def _(j):
            pltpu.make_async_copy(
                val_hbm.at[idxb.at[pl.ds(j * TAPS, TAPS)]], rows.at[j], gsem
            ).start()

    def compute(k, s):
        idxb, wgtb, rows, olo, ohi, gsem = bufs[s][:6]

        @pl.loop(0, CH)
        def _(j):
            pltpu.make_async_copy(
                val_hbm.at[idxb.at[pl.ds(j * TAPS, TAPS)]], rows.at[j], gsem
            ).wait()

            jbase = jnp.full((16,), j * TAPS, jnp.int32)

            def head_body(h, hb, ob):
                acc = [jnp.zeros((16,), jnp.float32) for _ in range(4)]
                for corner in range(4):
                    half = 2 * (corner // 2)
                    for p in range(P):
                        t = corner * 32 + h * P + p
                        w = plsc.load_gather(wgtb, [jbase + t])
                        pr = rows[j, t, ...]
                        ev = plsc.bitcast(pr << 16, jnp.float32)
                        od = plsc.bitcast(pr, jnp.float32)
                        acc[half] = acc[half] + w * ev
                        acc[half + 1] = acc[half + 1] + w * od
                ob[pl.ds(j * 128 + hb * 32, 16)] = acc[0] + acc[2]
                ob[pl.ds(j * 128 + hb * 32 + 16, 16)] = acc[1] + acc[3]

            @pl.loop(0, 4)
            def _(hb):
                head_body(hb, hb, olo)

            @pl.loop(0, 4)
            def _(hb):
                head_body(hb + 4, hb, ohi)

    start(in_copies(0, 0))
    wait(in_copies(0, 0))
    start_gath(0)
    start(in_copies(1, 1))

    @pl.loop(0, NCH // 2)
    def _(k2):
        k = 2 * k2
        wait(in_copies(k + 1, 1))
        start_gath(1)

        @pl.when(k >= 2)
        def _():
            wait(out_copies(k - 2, 0))

        compute(k, 0)
        start(out_copies(k, 0))

        @pl.when(k + 2 < NCH)
        def _():
            start(in_copies(k + 2, 0))
            wait(in_copies(k + 2, 0))
            start_gath(0)

        @pl.when(k >= 1)
        def _():
            wait(out_copies(k - 1, 1))

        compute(k + 1, 1)
        start(out_copies(k + 1, 1))

        @pl.when(k + 3 < NCH)
        def _():
            start(in_copies(k + 3, 1))

    if NCH % 2:
        wait(out_copies(NCH - 3, 0))
        compute(NCH - 1, 0)
        start(out_copies(NCH - 1, 0))
        wait(out_copies(NCH - 2, 1))
        wait(out_copies(NCH - 1, 0))
    else:
        wait(out_copies(NCH - 2, 0))
        wait(out_copies(NCH - 1, 1))


_sc_cp = pltpu.CompilerParams()
for _f, _v in (("needs_layout_passes", False), ("use_tc_tiling_on_sc", False)):
    if _f in pltpu.CompilerParams.__dataclass_fields__:
        _sc_cp = dataclasses.replace(_sc_cp, **{_f: _v})

_sc_call = pl.kernel(
    out_type=[
        jax.ShapeDtypeStruct((N * 128,), jnp.float32),
        jax.ShapeDtypeStruct((N * 128,), jnp.float32),
    ],
    compiler_params=_sc_cp,
    mesh=plsc.VectorSubcoreMesh(core_axis_name="c", subcore_axis_name="s"),
    scratch_types=[
        pltpu.VMEM((CH * TAPS,), jnp.int32),
        pltpu.VMEM((CH * TAPS,), jnp.int32),
        pltpu.VMEM((CH * TAPS,), jnp.float32),
        pltpu.VMEM((CH * TAPS,), jnp.float32),
        pltpu.VMEM((CH, TAPS, 16), jnp.uint32),
        pltpu.VMEM((CH, TAPS, 16), jnp.uint32),
        pltpu.VMEM((CH * 128,), jnp.float32),
        pltpu.VMEM((CH * 128,), jnp.float32),
        pltpu.VMEM((CH * 128,), jnp.float32),
        pltpu.VMEM((CH * 128,), jnp.float32),
        pltpu.SemaphoreType.DMA,
        pltpu.SemaphoreType.DMA,
        pltpu.SemaphoreType.DMA,
        pltpu.SemaphoreType.DMA,
        pltpu.SemaphoreType.DMA,
        pltpu.SemaphoreType.DMA,
    ],
)(_sc_bag)


def kernel(x, x_pos, W_off, b_off, W_aw, b_aw, W_val, b_val, W_out, b_out):
    x2 = x.reshape(N, D)
    xp2 = x_pos.reshape(N, D)
    wox = W_off[:, 0::2]
    box = b_off[0::2].reshape(1, 32)
    woy = W_off[:, 1::2]
    boy = b_off[1::2].reshape(1, 32)
    baw = b_aw.reshape(1, 32)
    ch = (np.arange(H)[:, None, None] * 32
          + np.arange(0, 32, 2)[None, None, :]
          + np.array([0, 1])[None, :, None])
    ce = jnp.asarray(ch[:, 0].reshape(-1))
    co = jnp.asarray(ch[:, 1].reshape(-1))
    wve, wvo = W_val[:, ce], W_val[:, co]
    bve, bvo = b_val[ce].reshape(1, 128), b_val[co].reshape(1, 128)
    m = jnp.asarray(np.kron(np.eye(H), np.ones((P, P))), jnp.float32)

    full = lambda r, c: pl.BlockSpec((r, c), lambda i: (0, 0))
    val, idx, wgt = pl.pallas_call(
        _proj_kernel,
        grid=(N // BQ,),
        in_specs=[
            pl.BlockSpec((BQ, D), lambda i: (i, 0)),
            pl.BlockSpec((BQ, D), lambda i: (i, 0)),
            full(D, 128), full(1, 128),
            full(D, 128), full(1, 128),
            full(D, 32), full(1, 32),
            full(D, 32), full(1, 32),
            full(D, 32), full(1, 32),
            full(32, 32),
        ],
        out_specs=[
            pl.BlockSpec((BQ, 128), lambda i: (i, 0)),
            pl.BlockSpec((BQ, 128), lambda i: (i, 0)),
            pl.BlockSpec((BQ, 128), lambda i: (i, 0)),
        ],
        out_shape=[
            jax.ShapeDtypeStruct((N, 128), jnp.uint32),
            jax.ShapeDtypeStruct((N, 128), jnp.int32),
            jax.ShapeDtypeStruct((N, 128), jnp.float32),
        ],
    )(x2, xp2, wve, bve, wvo, bvo, wox, box, woy, boy, W_aw, baw, m)

    lo, hi = _sc_call(
        val.reshape(N * H, 16), idx.reshape(N * TAPS), wgt.reshape(N * TAPS)
    )

    wp = W_out[_ROWS_PERM]
    out = pl.pallas_call(
        _out_kernel,
        grid=(N // BQ,),
        in_specs=[
            pl.BlockSpec((BQ, 128), lambda i: (i, 0)),
            pl.BlockSpec((BQ, 128), lambda i: (i, 0)),
            pl.BlockSpec((BQ, D), lambda i: (i, 0)),
            full(128, D), full(128, D), full(1, D),
        ],
        out_specs=pl.BlockSpec((BQ, D), lambda i: (i, 0)),
        out_shape=jax.ShapeDtypeStruct((N, D), jnp.float32),
    )(lo.reshape(N, 128), hi.reshape(N, 128), x2, wp[:128], wp[128:],
      b_out.reshape(1, D))
    return out.reshape(1, N, D)


# speedup vs baseline: 1.3603x; 1.0012x over previous
import dataclasses

import jax
import jax.numpy as jnp
import numpy as np
from jax import lax
from jax.experimental import pallas as pl
from jax.experimental.pallas import tpu as pltpu
from jax.experimental.pallas import tpu_sc as plsc

D = 256
H = 8
P = 4
GH = 200
GW = 200
N = GH * GW
TAPS = 4 * H * P
BQ = 2000
NW = 32
QPW = N // NW
CH = 25
NCH = QPW // CH

_PERM = np.concatenate([np.arange(0, 32, 2), np.arange(1, 32, 2)])
_ROWS_PERM = (np.arange(H)[:, None] * 32 + _PERM[None, :]).reshape(-1)


def _proj_kernel(x_ref, xp_ref, wve_ref, bve_ref, wvo_ref, bvo_ref,
                 wox_ref, box_ref, woy_ref, boy_ref, waw_ref, baw_ref, m_ref,
                 val_ref, idx_ref, wgt_ref):
    i = pl.program_id(0)
    xb = x_ref[...]
    q = xb + xp_ref[...]
    ve = jnp.dot(xb, wve_ref[...], preferred_element_type=jnp.float32) + bve_ref[...]
    vo = jnp.dot(xb, wvo_ref[...], preferred_element_type=jnp.float32) + bvo_ref[...]

    def rne_bf16(v):
        u = jax.lax.bitcast_convert_type(v, jnp.uint32)
        return (u + 0x7FFF + ((u >> 16) & 1)) >> 16

    val_ref[...] = rne_bf16(ve) | (rne_bf16(vo) << 16)
    ox = jnp.dot(q, wox_ref[...], preferred_element_type=jnp.float32) + box_ref[...]
    oy = jnp.dot(q, woy_ref[...], preferred_element_type=jnp.float32) + boy_ref[...]
    logits = jnp.dot(q, waw_ref[...], preferred_element_type=jnp.float32) + baw_ref[...]
    e = jnp.exp(logits)
    gs = jnp.dot(e, m_ref[...], preferred_element_type=jnp.float32)
    aw = e / gs
    qid = i * BQ + lax.broadcasted_iota(jnp.int32, (BQ, 32), 0)
    row = (qid // GW).astype(jnp.float32)
    col = (qid % GW).astype(jnp.float32)
    px = col + ox
    py = row + oy
    x0f = jnp.floor(px)
    y0f = jnp.floor(py)
    lx = px - x0f
    ly = py - y0f
    x0 = x0f.astype(jnp.int32)
    y0 = y0f.astype(jnp.int32)
    x1 = x0 + 1
    y1 = y0 + 1
    head = lax.broadcasted_iota(jnp.int32, (BQ, 32), 1) // P

    def tap(xi, yi, w):
        valid = (xi >= 0) & (xi < GW) & (yi >= 0) & (yi < GH)
        xc = jnp.clip(xi, 0, GW - 1)
        yc = jnp.clip(yi, 0, GH - 1)
        return (yc * GW + xc) * H + head, aw * w * valid.astype(jnp.float32)

    i00, w00 = tap(x0, y0, (1.0 - lx) * (1.0 - ly))
    i10, w10 = tap(x1, y0, lx * (1.0 - ly))
    i01, w01 = tap(x0, y1, (1.0 - lx) * ly)
    i11, w11 = tap(x1, y1, lx * ly)
    idx_ref[...] = jnp.concatenate([i00, i10, i01, i11], axis=1)
    wgt_ref[...] = jnp.concatenate([w00, w10, w01, w11], axis=1)


def _out_kernel(lo_ref, hi_ref, x_ref, wlo_ref, whi_ref, bout_ref, o_ref):
    o_ref[...] = (
        jnp.dot(lo_ref[...], wlo_ref[...], preferred_element_type=jnp.float32)
        + jnp.dot(hi_ref[...], whi_ref[...], preferred_element_type=jnp.float32)
        + bout_ref[...]
        + x_ref[...]
    )


def _sc_bag(val_hbm, idx_hbm, wgt_hbm, lo_hbm, hi_hbm,
            idxb0, idxb1, wgtb0, wgtb1, rows0, rows1,
            olo0, olo1, ohi0, ohi1,
            gsem0, gsem1, isem0, isem1, osem0, osem1):
    wid = lax.axis_index("s") * 2 + lax.axis_index("c")
    base = wid * QPW
    bufs = (
        (idxb0, wgtb0, rows0, olo0, ohi0, gsem0, isem0, osem0),
        (idxb1, wgtb1, rows1, olo1, ohi1, gsem1, isem1, osem1),
    )

    def in_copies(k, s):
        idxb, wgtb = bufs[s][0], bufs[s][1]
        isem = bufs[s][6]
        qb = base + k * CH
        return (
            pltpu.make_async_copy(
                idx_hbm.at[pl.ds(qb * TAPS, CH * TAPS)], idxb, isem),
            pltpu.make_async_copy(
                wgt_hbm.at[pl.ds(qb * TAPS, CH * TAPS)], wgtb, isem),
        )

    def out_copies(k, s):
        olo, ohi = bufs[s][3], bufs[s][4]
        osem = bufs[s][7]
        qb = base + k * CH
        return (
            pltpu.make_async_copy(
                olo, lo_hbm.at[pl.ds(qb * 128, CH * 128)], osem),
            pltpu.make_async_copy(
                ohi, hi_hbm.at[pl.ds(qb * 128, CH * 128)], osem),
        )

    def start(copies):
        for c in copies:
            c.start()

    def wait(copies):
        for c in copies:
            c.wait()

    def start_gath(s):
        idxb, rows, gsem = bufs[s][0], bufs[s][2], bufs[s][5]

        @pl.loop(0, CH)
        def _(j):
            pltpu.make_async_copy(
                val_hbm.at[idxb.at[pl.ds(j * TAPS, TAPS)]], rows.at[j], gsem
            ).start()

    def compute(k, s):
        idxb, wgtb, rows, olo, ohi, gsem = bufs[s][:6]

        @pl.loop(0, CH)
        def _(j):
            pltpu.make_async_copy(
                val_hbm.at[idxb.at[pl.ds(j * TAPS, TAPS)]], rows.at[j], gsem
            ).wait()

            jbase = jnp.full((16,), j * TAPS, jnp.int32)

            def head_body(h, hb, ob):
                acc = [jnp.zeros((16,), jnp.float32) for _ in range(4)]
                for corner in range(4):
                    half = 2 * (corner // 2)
                    for p in range(P):
                        t = corner * 32 + h * P + p
                        w = plsc.load_gather(wgtb, [jbase + t])
                        pr = rows[j, t, ...]
                        ev = plsc.bitcast(pr << 16, jnp.float32)
                        od = plsc.bitcast(pr, jnp.float32)
                        acc[half] = acc[half] + w * ev
                        acc[half + 1] = acc[half + 1] + w * od
                ob[pl.ds(j * 128 + hb * 32, 16)] = acc[0] + acc[2]
                ob[pl.ds(j * 128 + hb * 32 + 16, 16)] = acc[1] + acc[3]

            @pl.loop(0, 4)
            def _(hb):
                head_body(hb, hb, olo)

            @pl.loop(0, 4)
            def _(hb):
                head_body(hb + 4, hb, ohi)

    start(in_copies(0, 0))
    wait(in_copies(0, 0))
    start_gath(0)
    start(in_copies(1, 1))

    @pl.loop(0, NCH // 2)
    def _(k2):
        k = 2 * k2
        wait(in_copies(k + 1, 1))
        start_gath(1)

        @pl.when(k >= 2)
        def _():
            wait(out_copies(k - 2, 0))

        compute(k, 0)
        start(out_copies(k, 0))

        @pl.when(k + 2 < NCH)
        def _():
            start(in_copies(k + 2, 0))
            wait(in_copies(k + 2, 0))
            start_gath(0)

        @pl.when(k >= 1)
        def _():
            wait(out_copies(k - 1, 1))

        compute(k + 1, 1)
        start(out_copies(k + 1, 1))

        @pl.when(k + 3 < NCH)
        def _():
            start(in_copies(k + 3, 1))

    if NCH % 2:
        wait(out_copies(NCH - 3, 0))
        compute(NCH - 1, 0)
        start(out_copies(NCH - 1, 0))
        wait(out_copies(NCH - 2, 1))
        wait(out_copies(NCH - 1, 0))
    else:
        wait(out_copies(NCH - 2, 0))
        wait(out_copies(NCH - 1, 1))


_sc_cp = pltpu.CompilerParams()
for _f, _v in (("needs_layout_passes", False), ("use_tc_tiling_on_sc", False)):
    if _f in pltpu.CompilerParams.__dataclass_fields__:
        _sc_cp = dataclasses.replace(_sc_cp, **{_f: _v})

_sc_call = pl.kernel(
    out_type=[
        jax.ShapeDtypeStruct((N * 128,), jnp.float32),
        jax.ShapeDtypeStruct((N * 128,), jnp.float32),
    ],
    compiler_params=_sc_cp,
    mesh=plsc.VectorSubcoreMesh(core_axis_name="c", subcore_axis_name="s"),
    scratch_types=[
        pltpu.VMEM((CH * TAPS,), jnp.int32),
        pltpu.VMEM((CH * TAPS,), jnp.int32),
        pltpu.VMEM((CH * TAPS,), jnp.float32),
        pltpu.VMEM((CH * TAPS,), jnp.float32),
        pltpu.VMEM((CH, TAPS, 16), jnp.uint32),
        pltpu.VMEM((CH, TAPS, 16), jnp.uint32),
        pltpu.VMEM((CH * 128,), jnp.float32),
        pltpu.VMEM((CH * 128,), jnp.float32),
        pltpu.VMEM((CH * 128,), jnp.float32),
        pltpu.VMEM((CH * 128,), jnp.float32),
        pltpu.SemaphoreType.DMA,
        pltpu.SemaphoreType.DMA,
        pltpu.SemaphoreType.DMA,
        pltpu.SemaphoreType.DMA,
        pltpu.SemaphoreType.DMA,
        pltpu.SemaphoreType.DMA,
    ],
)(_sc_bag)


def kernel(x, x_pos, W_off, b_off, W_aw, b_aw, W_val, b_val, W_out, b_out):
    x2 = x.reshape(N, D)
    xp2 = x_pos.reshape(N, D)
    wox = W_off[:, 0::2]
    box = b_off[0::2].reshape(1, 32)
    woy = W_off[:, 1::2]
    boy = b_off[1::2].reshape(1, 32)
    baw = b_aw.reshape(1, 32)
    ch = (np.arange(H)[:, None, None] * 32
          + np.arange(0, 32, 2)[None, None, :]
          + np.array([0, 1])[None, :, None])
    ce = jnp.asarray(ch[:, 0].reshape(-1))
    co = jnp.asarray(ch[:, 1].reshape(-1))
    wve, wvo = W_val[:, ce], W_val[:, co]
    bve, bvo = b_val[ce].reshape(1, 128), b_val[co].reshape(1, 128)
    m = jnp.asarray(np.kron(np.eye(H), np.ones((P, P))), jnp.float32)

    full = lambda r, c: pl.BlockSpec((r, c), lambda i: (0, 0))
    val, idx, wgt = pl.pallas_call(
        _proj_kernel,
        grid=(N // BQ,),
        in_specs=[
            pl.BlockSpec((BQ, D), lambda i: (i, 0)),
            pl.BlockSpec((BQ, D), lambda i: (i, 0)),
            full(D, 128), full(1, 128),
            full(D, 128), full(1, 128),
            full(D, 32), full(1, 32),
            full(D, 32), full(1, 32),
            full(D, 32), full(1, 32),
            full(32, 32),
        ],
        out_specs=[
            pl.BlockSpec((BQ, 128), lambda i: (i, 0)),
            pl.BlockSpec((BQ, 128), lambda i: (i, 0)),
            pl.BlockSpec((BQ, 128), lambda i: (i, 0)),
        ],
        out_shape=[
            jax.ShapeDtypeStruct((N, 128), jnp.uint32),
            jax.ShapeDtypeStruct((N, 128), jnp.int32),
            jax.ShapeDtypeStruct((N, 128), jnp.float32),
        ],
    )(x2, xp2, wve, bve, wvo, bvo, wox, box, woy, boy, W_aw, baw, m)

    lo, hi = _sc_call(
        val.reshape(N * H, 16), idx.reshape(N * TAPS), wgt.reshape(N * TAPS)
    )

    wp = W_out[_ROWS_PERM]
    out = pl.pallas_call(
        _out_kernel,
        grid=(N // BQ,),
        in_specs=[
            pl.BlockSpec((BQ, 128), lambda i: (i, 0)),
            pl.BlockSpec((BQ, 128), lambda i: (i, 0)),
            pl.BlockSpec((BQ, D), lambda i: (i, 0)),
            full(128, D), full(128, D), full(1, D),
        ],
        out_specs=pl.BlockSpec((BQ, D), lambda i: (i, 0)),
        out_shape=jax.ShapeDtypeStruct((N, D), jnp.float32),
    )(lo.reshape(N, 128), hi.reshape(N, 128), x2, wp[:128], wp[128:],
      b_out.reshape(1, D))
    return out.reshape(1, N, D)
